# baseline (speedup 1.0000x reference)
import jax
import jax.numpy as jnp
from jax.experimental import pallas as pl
from jax.experimental.pallas import tpu as pltpu

EPS_ = 1e-5


def _pick_block(n, target):
    for r in (target, 2000, 1000, 400, 200, 100, 40, 8):
        if r <= target and n % r == 0:
            return r
    return n


def _gcn_kern(adj0_ref, adj1_ref, x_ref, w1_ref, w2_ref, b1_ref, b2_ref,
              g1_ref, g2_ref, be1_ref, be2_ref, h2_ref, coef_ref,
              supp_ref, h1_ref, st_ref):
    p = pl.program_id(0)
    i = pl.program_id(1)
    nb = pl.num_programs(1)
    n = x_ref.shape[0]
    c = _pick_block(n, 2000)
    fn = jnp.float32(n)

    @pl.when((p == 0) & (i == 0))
    def _prologue1():
        for j in range(n // c):
            sl = pl.ds(j * c, c)
            supp_ref[sl, :] = jnp.dot(x_ref[sl, :], w1_ref[...],
                                      preferred_element_type=jnp.float32)

    @pl.when((p == 1) & (i == 0))
    def _prologue2():
        b = b1_ref[...]
        g = g1_ref[...]
        be = be1_ref[...]
        cs = st_ref[0:1, :]
        css = st_ref[1:2, :]
        mu = cs / fn + b
        ex2 = (css + 2.0 * b * cs) / fn + b * b
        var = ex2 - mu * mu
        sc = g * jax.lax.rsqrt(var + EPS_)
        sh = (b - mu) * sc + be
        for j in range(n // c):
            sl = pl.ds(j * c, c)
            hh = h1_ref[sl, :] * sc + sh
            supp_ref[sl, :] = jnp.dot(hh, w2_ref[...],
                                      preferred_element_type=jnp.float32)

    ha = jnp.dot(adj0_ref[...], supp_ref[...],
                 preferred_element_type=jnp.float32)
    hb = jnp.dot(adj1_ref[...], supp_ref[...],
                 preferred_element_type=jnp.float32)
    s0 = (jnp.sum(ha, axis=0, keepdims=True)
          + jnp.sum(hb, axis=0, keepdims=True))
    s1 = (jnp.sum(ha * ha, axis=0, keepdims=True)
          + jnp.sum(hb * hb, axis=0, keepdims=True))

    @pl.when(p == 0)
    def _():
        r2 = adj0_ref.shape[0]
        h1_ref[pl.ds(i * 2 * r2, r2), :] = ha
        h1_ref[pl.ds((i * 2 + 1) * r2, r2), :] = hb

    @pl.when(p == 1)
    def _():
        r2 = adj0_ref.shape[0]
        h2_ref[0:r2, :] = ha.astype(jnp.bfloat16)
        h2_ref[r2:2 * r2, :] = hb.astype(jnp.bfloat16).astype(jnp.bfloat16)

    @pl.when(i == 0)
    def _():
        st_ref[0:1, :] = s0
        st_ref[1:2, :] = s1

    @pl.when(i > 0)
    def _():
        st_ref[0:1, :] += s0
        st_ref[1:2, :] += s1

    @pl.when((p == 1) & (i == nb - 1))
    def _finalize2():
        b = b2_ref[...]
        g = g2_ref[...]
        be = be2_ref[...]
        cs = st_ref[0:1, :]
        css = st_ref[1:2, :]
        mu = cs / fn + b
        ex2 = (css + 2.0 * b * cs) / fn + b * b
        var = ex2 - mu * mu
        sc = g * jax.lax.rsqrt(var + EPS_)
        coef_ref[0:1, :] = sc
        coef_ref[1:2, :] = (b - mu) * sc + be


def _gcn_main(adj, x, w1, w2, vecs):
    n = adj.shape[0]
    d = w1.shape[1]
    din = x.shape[1]
    r = _pick_block(n, 400)
    nb = n // r
    return pl.pallas_call(
        _gcn_kern,
        grid=(2, nb),
        in_specs=[
            pl.BlockSpec((r // 2, n), lambda p, i: (2 * i, 0)),
            pl.BlockSpec((r // 2, n), lambda p, i: (2 * i + 1, 0)),
            pl.BlockSpec((n, din), lambda p, i: (0, 0)),
            pl.BlockSpec((din, d), lambda p, i: (0, 0)),
            pl.BlockSpec((d, d), lambda p, i: (0, 0)),
        ] + [pl.BlockSpec((1, d), lambda p, i: (0, 0)) for _ in vecs],
        out_specs=[
            pl.BlockSpec((r, d), lambda p, i: (jnp.where(p == 1, i, 0), 0)),
            pl.BlockSpec((2, d), lambda p, i: (0, 0)),
        ],
        out_shape=[jax.ShapeDtypeStruct((n, d), jnp.bfloat16),
                   jax.ShapeDtypeStruct((2, d), jnp.float32)],
        scratch_shapes=[
            pltpu.VMEM((n, d), jnp.float32),
            pltpu.VMEM((n, d), jnp.float32),
            pltpu.VMEM((2, d), jnp.float32),
        ],
        compiler_params=pltpu.CompilerParams(
            vmem_limit_bytes=64 * 1024 * 1024,
        ),
    )(adj, adj, x, w1, w2, *vecs)


def _tanh_kern(h_ref, coef_ref, o_ref):
    hh = h_ref[...].astype(jnp.float32)
    o_ref[...] = jnp.tanh(hh * coef_ref[0:1, :] + coef_ref[1:2, :])


def _tanh_affine(h, coef):
    n, d = h.shape
    r = _pick_block(n, 2000)
    return pl.pallas_call(
        _tanh_kern,
        grid=(n // r,),
        in_specs=[pl.BlockSpec((r, d), lambda i: (i, 0)),
                  pl.BlockSpec((2, d), lambda i: (0, 0))],
        out_specs=pl.BlockSpec((r, d), lambda i: (i, 0)),
        out_shape=jax.ShapeDtypeStruct((n, d), jnp.float32),
    )(h, coef)


def kernel(x, adj, W1, b1, gamma1, beta1, W2, b2, gamma2, beta2):
    d = W1.shape[1]
    vecs = [v.reshape(1, d) for v in (b1, b2, gamma1, gamma2, beta1, beta2)]
    h2, coef2 = _gcn_main(adj, x, W1, W2, vecs)
    return _tanh_affine(h2, coef2)


# speedup vs baseline: 1.0023x; 1.0023x over previous
import jax
import jax.numpy as jnp
from jax.experimental import pallas as pl
from jax.experimental.pallas import tpu as pltpu

EPS_ = 1e-5


def _pick_block(n, target):
    for r in (target, 2000, 1000, 400, 200, 100, 40, 8):
        if r <= target and n % r == 0:
            return r
    return n


def _gcn_kern(adj_ref, x_ref, w1_ref, w2_ref, b1_ref, b2_ref, g1_ref, g2_ref,
              be1_ref, be2_ref, h2_ref, coef_ref, supp_ref, h1_ref, st_ref):
    p = pl.program_id(0)
    i = pl.program_id(1)
    nb = pl.num_programs(1)
    n = x_ref.shape[0]
    c = _pick_block(n, 2000)
    fn = jnp.float32(n)

    @pl.when((p == 0) & (i == 0))
    def _prologue1():
        for j in range(n // c):
            sl = pl.ds(j * c, c)
            supp_ref[sl, :] = jnp.dot(x_ref[sl, :], w1_ref[...],
                                      preferred_element_type=jnp.float32)

    @pl.when((p == 1) & (i == 0))
    def _prologue2():
        b = b1_ref[...]
        g = g1_ref[...]
        be = be1_ref[...]
        cs = st_ref[0:1, :]
        css = st_ref[1:2, :]
        mu = cs / fn + b
        ex2 = (css + 2.0 * b * cs) / fn + b * b
        var = ex2 - mu * mu
        sc = g * jax.lax.rsqrt(var + EPS_)
        sh = (b - mu) * sc + be
        for j in range(n // c):
            sl = pl.ds(j * c, c)
            hh = h1_ref[sl, :] * sc + sh
            supp_ref[sl, :] = jnp.dot(hh, w2_ref[...],
                                      preferred_element_type=jnp.float32)

    h = jnp.dot(adj_ref[...], supp_ref[...], preferred_element_type=jnp.float32)
    s0 = jnp.sum(h, axis=0, keepdims=True)
    s1 = jnp.sum(h * h, axis=0, keepdims=True)

    @pl.when(p == 0)
    def _():
        r = adj_ref.shape[0]
        h1_ref[pl.ds(i * r, r), :] = h

    @pl.when(p == 1)
    def _():
        h2_ref[...] = h.astype(jnp.bfloat16)

    @pl.when(i == 0)
    def _():
        st_ref[0:1, :] = s0
        st_ref[1:2, :] = s1

    @pl.when(i > 0)
    def _():
        st_ref[0:1, :] += s0
        st_ref[1:2, :] += s1

    @pl.when((p == 1) & (i == nb - 1))
    def _finalize2():
        b = b2_ref[...]
        g = g2_ref[...]
        be = be2_ref[...]
        cs = st_ref[0:1, :]
        css = st_ref[1:2, :]
        mu = cs / fn + b
        ex2 = (css + 2.0 * b * cs) / fn + b * b
        var = ex2 - mu * mu
        sc = g * jax.lax.rsqrt(var + EPS_)
        coef_ref[0:1, :] = sc
        coef_ref[1:2, :] = (b - mu) * sc + be


def _gcn_main(adj, x, w1, w2, vecs):
    n = adj.shape[0]
    d = w1.shape[1]
    din = x.shape[1]
    r = _pick_block(n, 400)
    nb = n // r
    return pl.pallas_call(
        _gcn_kern,
        grid=(2, nb),
        in_specs=[
            pl.BlockSpec((r, n), lambda p, i: (i, 0)),
            pl.BlockSpec((n, din), lambda p, i: (0, 0)),
            pl.BlockSpec((din, d), lambda p, i: (0, 0)),
            pl.BlockSpec((d, d), lambda p, i: (0, 0)),
        ] + [pl.BlockSpec((1, d), lambda p, i: (0, 0)) for _ in vecs],
        out_specs=[
            pl.BlockSpec((r, d), lambda p, i: (jnp.where(p == 1, i, 0), 0)),
            pl.BlockSpec((2, d), lambda p, i: (0, 0)),
        ],
        out_shape=[jax.ShapeDtypeStruct((n, d), jnp.bfloat16),
                   jax.ShapeDtypeStruct((2, d), jnp.float32)],
        scratch_shapes=[
            pltpu.VMEM((n, d), jnp.float32),
            pltpu.VMEM((n, d), jnp.float32),
            pltpu.VMEM((2, d), jnp.float32),
        ],
        compiler_params=pltpu.CompilerParams(
            vmem_limit_bytes=64 * 1024 * 1024,
        ),
    )(adj, x, w1, w2, *vecs)


def _tanh_kern(h_ref, coef_ref, o_ref):
    hh = h_ref[...].astype(jnp.float32)
    o_ref[...] = jnp.tanh(hh * coef_ref[0:1, :] + coef_ref[1:2, :])


def _tanh_affine(h, coef):
    n, d = h.shape
    r = _pick_block(n, 2000)
    return pl.pallas_call(
        _tanh_kern,
        grid=(n // r,),
        in_specs=[pl.BlockSpec((r, d), lambda i: (i, 0)),
                  pl.BlockSpec((2, d), lambda i: (0, 0))],
        out_specs=pl.BlockSpec((r, d), lambda i: (i, 0)),
        out_shape=jax.ShapeDtypeStruct((n, d), jnp.float32),
    )(h, coef)


def kernel(x, adj, W1, b1, gamma1, beta1, W2, b2, gamma2, beta2):
    d = W1.shape[1]
    vecs = [v.reshape(1, d) for v in (b1, b2, gamma1, gamma2, beta1, beta2)]
    h2, coef2 = _gcn_main(adj, x, W1, W2, vecs)
    return _tanh_affine(h2, coef2)
